# baseline (speedup 1.0000x reference)
import functools

import jax
import jax.numpy as jnp
from jax import lax
from jax.experimental import pallas as pl
from jax.experimental.pallas import tpu as pltpu
from jax.experimental.pallas import tpu_sc as plsc

V = 1000000
D = 32
B = 16384
K = 20

NC = 2
NS = 16
NW = NC * NS

NFULL = V // 128
VREM = V - NFULL * 128
VPAD = (NFULL + 1) * 128
RW = 48
LINW = VPAD * RW
GSZ = 768
GN = NFULL * 128 // GSZ
GRP_PER_W = GN // NW + 1

NB = B // NW
C = 64
NCHUNK = NB // C


def _rel_body(tc_h, to_h, tailc_h, tailo_h, linc_h, lino_h,
              stga, stgb, outa, outb, semra, semrb, semw0, semw1):
    i32 = jnp.int32
    wid = lax.axis_index("s") * NC + lax.axis_index("c")
    iota_rw = lax.broadcasted_iota(i32, (16,), 0) * RW
    iota32 = lax.broadcasted_iota(i32, (16,), 0) * 32

    def transpose(stg, ob):
        @plsc.parallel_loop(0, GSZ // 16, unroll=4)
        def gbody(g):
            base = iota_rw + g * (16 * RW)
            for d in range(D):
                vec = stg[d, pl.ds(g * 16, 16)]
                plsc.store_scatter(ob, [base + d], vec)

    def do_table(t_h, tail_h, lin_h):
        def fire_reads(grp, stg, semr):
            for q in range(4):
                pltpu.async_copy(
                    t_h.at[pl.ds(q * 8, 8), pl.ds(grp * GSZ, GSZ)],
                    stg.at[pl.ds(q * 8, 8), :], semr)

        def wait_reads(stg, semr):
            for q in range(4):
                pltpu.make_async_copy(
                    t_h.at[pl.ds(0, 8), pl.ds(0, GSZ)],
                    stg.at[pl.ds(q * 8, 8), :], semr).wait()

        def fire_write(grp, ob, semw):
            pltpu.async_copy(ob, lin_h.at[pl.ds(grp * GSZ * RW, GSZ * RW)], semw)

        def drain_write(ob, semw):
            pltpu.make_async_copy(lin_h.at[pl.ds(0, GSZ * RW)], ob, semw).wait()

        def step(i, grp, stg, semr, ob, semw, pf_grp, pf_stg, pf_semr):
            @pl.when(pf_grp < GN)
            def _():
                fire_reads(pf_grp, pf_stg, pf_semr)

            @pl.when(grp < GN)
            def _():
                wait_reads(stg, semr)

                @pl.when(i > 0)
                def _():
                    drain_write(ob, semw)

                transpose(stg, ob)
                fire_write(grp, ob, semw)

        fire_reads(wid, stga, semra)

        def pair(i, _):
            ga = wid + 64 * i
            gb = ga + 32
            step(i, ga, stga, semra, outa, semw0, gb, stgb, semrb)
            step(i, gb, stgb, semrb, outb, semw1, ga + 64, stga, semra)
            return 0

        lax.fori_loop(0, (GRP_PER_W + 1) // 2, pair, 0)
        drain_write(outa, semw0)
        drain_write(outb, semw1)

        @pl.when(wid == 0)
        def _():
            pltpu.sync_copy(tail_h, outb.at[pl.ds(0, VREM * D)])
            for g in range(VREM // 16):
                for d in range(D):
                    vec = plsc.load_gather(outb, [iota32 + (g * 512 + d)])
                    plsc.store_scatter(outa, [iota_rw + (g * 16 * RW + d)], vec)
            pltpu.sync_copy(outa.at[pl.ds(0, VREM * RW)],
                            lin_h.at[pl.ds(NFULL * 128 * RW, VREM * RW)])

    do_table(tc_h, tailc_h, linc_h)
    do_table(to_h, tailo_h, lino_h)


@jax.jit
def _sc_relayout(tc, to, tailc, tailo):
    mesh = plsc.VectorSubcoreMesh(core_axis_name="c", subcore_axis_name="s")
    f32 = jnp.float32
    return pl.kernel(
        _rel_body,
        out_type=(
            jax.ShapeDtypeStruct((LINW,), f32),
            jax.ShapeDtypeStruct((LINW,), f32),
        ),
        mesh=mesh,
        compiler_params=pltpu.CompilerParams(use_tc_tiling_on_sc=True,
                                             needs_layout_passes=False),
        scratch_types=[
            pltpu.VMEM((D, GSZ), f32),
            pltpu.VMEM((D, GSZ), f32),
            pltpu.VMEM((GSZ * RW,), f32),
            pltpu.VMEM((GSZ * RW,), f32),
            pltpu.SemaphoreType.DMA,
            pltpu.SemaphoreType.DMA,
            pltpu.SemaphoreType.DMA,
            pltpu.SemaphoreType.DMA,
        ],
    )(tc, to, tailc, tailo)


def _sc_body(emb_c, emb_o, cidx_h, oidx_h, nidx_h, uo_h, ng_h,
             cidx_v, oidx_v, nidx_v, crows, orows, nrows, uo_v, ng_v, sem):
    wid = lax.axis_index("s") * NC + lax.axis_index("c")
    pltpu.sync_copy(cidx_h.at[pl.ds(wid * NB, NB)], cidx_v)
    pltpu.sync_copy(oidx_h.at[pl.ds(wid * NB, NB)], oidx_v)
    pltpu.sync_copy(nidx_h.at[pl.ds(wid * NB * K, NB * K)], nidx_v)

    def chunk_body(ch, _):
        cbase = ch * C
        cps = [
            pltpu.async_copy(emb_c.at[cidx_v.at[pl.ds(cbase, C)]], crows, sem),
            pltpu.async_copy(emb_o.at[oidx_v.at[pl.ds(cbase, C)]], orows, sem),
        ]
        for j in range(K):
            cps.append(pltpu.async_copy(
                emb_o.at[nidx_v.at[pl.ds(cbase * K + j * C, C)]],
                nrows.at[pl.ds(j * C, C)], sem))
        for cp in cps:
            cp.wait()

        def bbody(b, _):
            c0 = crows[b, pl.ds(0, 16)]
            c1 = crows[b, pl.ds(16, 16)]
            o0 = orows[b, pl.ds(0, 16)]
            o1 = orows[b, pl.ds(16, 16)]
            uo_v[pl.ds(b * 16, 16)] = c0 * o0 + c1 * o1
            base = b * K
            a0 = nrows[base, pl.ds(0, 16)]
            a1 = nrows[base, pl.ds(16, 16)]
            for k in range(1, K):
                a0 = a0 + nrows[base + k, pl.ds(0, 16)]
                a1 = a1 + nrows[base + k, pl.ds(16, 16)]
            ng_v[pl.ds(b * 16, 16)] = c0 * a0 + c1 * a1
            return 0

        lax.fori_loop(0, C, bbody, 0)
        out_base = (wid * NB + cbase) * 16
        pltpu.sync_copy(uo_v, uo_h.at[pl.ds(out_base, C * 16)])
        pltpu.sync_copy(ng_v, ng_h.at[pl.ds(out_base, C * 16)])
        return 0

    lax.fori_loop(0, NCHUNK, chunk_body, 0)


@jax.jit
def _sc_partials(cidx, oidx, nidx, emb_c_lin, emb_o_lin):
    mesh = plsc.VectorSubcoreMesh(core_axis_name="c", subcore_axis_name="s")
    f32 = jnp.float32
    i32 = jnp.int32
    return pl.kernel(
        _sc_body,
        out_type=(
            jax.ShapeDtypeStruct((B * 16,), f32),
            jax.ShapeDtypeStruct((B * 16,), f32),
        ),
        mesh=mesh,
        compiler_params=pltpu.CompilerParams(use_tc_tiling_on_sc=False),
        scratch_types=[
            pltpu.VMEM((NB,), i32),
            pltpu.VMEM((NB,), i32),
            pltpu.VMEM((NB * K,), i32),
            pltpu.VMEM((C, RW), f32),
            pltpu.VMEM((C, RW), f32),
            pltpu.VMEM((K * C, RW), f32),
            pltpu.VMEM((C * 16,), f32),
            pltpu.VMEM((C * 16,), f32),
            pltpu.SemaphoreType.DMA,
        ],
    )(emb_c_lin, emb_o_lin, cidx, oidx, nidx)


def _tc_body(uo_ref, ng_ref, out_ref):
    uo = uo_ref[...]
    ng = ng_ref[...]
    gi = lax.broadcasted_iota(jnp.int32, (128, 8), 0) // 16
    gj = lax.broadcasted_iota(jnp.int32, (128, 8), 1)
    g = (gi == gj).astype(jnp.float32)
    dn = (((1,), (0,)), ((), ()))
    uos = lax.dot_general(uo, g, dn, preferred_element_type=jnp.float32)
    ngs = lax.dot_general(ng, g, dn, preferred_element_type=jnp.float32)

    def logsig(t):
        return jnp.minimum(t, 0.0) - jnp.log1p(jnp.exp(-jnp.abs(t)))

    loss = logsig(uos) + logsig(-ngs)
    out_ref[0, 0] = -jnp.sum(loss) / jnp.float32(B)


@jax.jit
def _tc_loss(uo2d, ng2d):
    return pl.pallas_call(
        _tc_body,
        out_shape=jax.ShapeDtypeStruct((1, 1), jnp.float32),
        out_specs=pl.BlockSpec(memory_space=pltpu.SMEM),
    )(uo2d, ng2d)


def kernel(center, outside, negative, emb_center, emb_outside):
    cidx = center.reshape(B)
    oidx = outside.reshape(B)
    nidx = negative.reshape(B * K)
    lin_c, lin_o = _sc_relayout(
        emb_center.T, emb_outside.T,
        emb_center[NFULL * 128:].reshape(VREM * D),
        emb_outside[NFULL * 128:].reshape(VREM * D))
    uo, ng = _sc_partials(cidx, oidx, nidx,
                          lin_c.reshape(VPAD, RW), lin_o.reshape(VPAD, RW))
    out = _tc_loss(uo.reshape(B * 16 // 128, 128), ng.reshape(B * 16 // 128, 128))
    return out[0, 0]


# speedup vs baseline: 1.0646x; 1.0646x over previous
import functools

import jax
import jax.numpy as jnp
from jax import lax
from jax.experimental import pallas as pl
from jax.experimental.pallas import tpu as pltpu
from jax.experimental.pallas import tpu_sc as plsc

V = 1000000
D = 32
B = 16384
K = 20

NC = 2
NS = 16
NW = NC * NS

NFULL = V // 128
VREM = V - NFULL * 128
VPAD = (NFULL + 1) * 128
RW = 48
LINW = VPAD * RW
GSZ = 512
GN = NFULL * 128 // GSZ
GRP_PER_W = GN // NW + 1

NB = B // NW
C = 64
NCHUNK = NB // C


def _rel_body(tc_h, to_h, tailc_h, tailo_h, linc_h, lino_h,
              stga, stgb, outa, outb, semra, semrb, semw0, semw1):
    i32 = jnp.int32
    wid = lax.axis_index("s") * NC + lax.axis_index("c")
    iota_rw = lax.broadcasted_iota(i32, (16,), 0) * RW
    iota32 = lax.broadcasted_iota(i32, (16,), 0) * 32

    def transpose(stg, ob):
        @plsc.parallel_loop(0, GSZ // 16, unroll=2)
        def gbody(g):
            base = iota_rw + g * (16 * RW)
            for d in range(D):
                vec = stg[d, pl.ds(g * 16, 16)]
                plsc.store_scatter(ob, [base + d], vec)

    def do_table(t_h, tail_h, lin_h):
        def fire_reads(grp, stg, semr):
            for q in range(4):
                pltpu.async_copy(
                    t_h.at[pl.ds(q * 8, 8), pl.ds(grp * GSZ, GSZ)],
                    stg.at[pl.ds(q * 8, 8), :], semr)

        def wait_reads(stg, semr):
            for q in range(4):
                pltpu.make_async_copy(
                    t_h.at[pl.ds(0, 8), pl.ds(0, GSZ)],
                    stg.at[pl.ds(q * 8, 8), :], semr).wait()

        def fire_write(grp, ob, semw):
            pltpu.async_copy(ob, lin_h.at[pl.ds(grp * GSZ * RW, GSZ * RW)], semw)

        def drain_write(ob, semw):
            pltpu.make_async_copy(lin_h.at[pl.ds(0, GSZ * RW)], ob, semw).wait()

        def step(i, grp, stg, semr, ob, semw, pf_grp, pf_stg, pf_semr):
            @pl.when(pf_grp < GN)
            def _():
                fire_reads(pf_grp, pf_stg, pf_semr)

            @pl.when(grp < GN)
            def _():
                wait_reads(stg, semr)

                @pl.when(i > 0)
                def _():
                    drain_write(ob, semw)

                transpose(stg, ob)
                fire_write(grp, ob, semw)

        fire_reads(wid, stga, semra)

        def pair(i, _):
            ga = wid + 64 * i
            gb = ga + 32
            step(i, ga, stga, semra, outa, semw0, gb, stgb, semrb)
            step(i, gb, stgb, semrb, outb, semw1, ga + 64, stga, semra)
            return 0

        lax.fori_loop(0, (GRP_PER_W + 1) // 2, pair, 0)
        drain_write(outa, semw0)
        drain_write(outb, semw1)

        @pl.when(wid == 0)
        def _():
            pltpu.sync_copy(tail_h, outb.at[pl.ds(0, VREM * D)])
            for g in range(VREM // 16):
                for d in range(D):
                    vec = plsc.load_gather(outb, [iota32 + (g * 512 + d)])
                    plsc.store_scatter(outa, [iota_rw + (g * 16 * RW + d)], vec)
            pltpu.sync_copy(outa.at[pl.ds(0, VREM * RW)],
                            lin_h.at[pl.ds(NFULL * 128 * RW, VREM * RW)])

    do_table(tc_h, tailc_h, linc_h)
    do_table(to_h, tailo_h, lino_h)


@jax.jit
def _sc_relayout(tc, to, tailc, tailo):
    mesh = plsc.VectorSubcoreMesh(core_axis_name="c", subcore_axis_name="s")
    f32 = jnp.float32
    return pl.kernel(
        _rel_body,
        out_type=(
            jax.ShapeDtypeStruct((LINW,), f32),
            jax.ShapeDtypeStruct((LINW,), f32),
        ),
        mesh=mesh,
        compiler_params=pltpu.CompilerParams(use_tc_tiling_on_sc=True,
                                             needs_layout_passes=False),
        scratch_types=[
            pltpu.VMEM((D, GSZ), f32),
            pltpu.VMEM((D, GSZ), f32),
            pltpu.VMEM((GSZ * RW,), f32),
            pltpu.VMEM((GSZ * RW,), f32),
            pltpu.SemaphoreType.DMA,
            pltpu.SemaphoreType.DMA,
            pltpu.SemaphoreType.DMA,
            pltpu.SemaphoreType.DMA,
        ],
    )(tc, to, tailc, tailo)


def _sc_body(emb_c, emb_o, cidx_h, oidx_h, nidx_h, uo_h, ng_h,
             cidx_v, oidx_v, nidx_v, crows, orows, nrows, uo_v, ng_v, sem):
    wid = lax.axis_index("s") * NC + lax.axis_index("c")
    pltpu.sync_copy(cidx_h.at[pl.ds(wid * NB, NB)], cidx_v)
    pltpu.sync_copy(oidx_h.at[pl.ds(wid * NB, NB)], oidx_v)
    pltpu.sync_copy(nidx_h.at[pl.ds(wid * NB * K, NB * K)], nidx_v)

    def chunk_body(ch, _):
        cbase = ch * C
        cps = [
            pltpu.async_copy(emb_c.at[cidx_v.at[pl.ds(cbase, C)]], crows, sem),
            pltpu.async_copy(emb_o.at[oidx_v.at[pl.ds(cbase, C)]], orows, sem),
        ]
        for j in range(K):
            cps.append(pltpu.async_copy(
                emb_o.at[nidx_v.at[pl.ds(cbase * K + j * C, C)]],
                nrows.at[pl.ds(j * C, C)], sem))
        for cp in cps:
            cp.wait()

        def bbody(b, _):
            c0 = crows[b, pl.ds(0, 16)]
            c1 = crows[b, pl.ds(16, 16)]
            o0 = orows[b, pl.ds(0, 16)]
            o1 = orows[b, pl.ds(16, 16)]
            uo_v[pl.ds(b * 16, 16)] = c0 * o0 + c1 * o1
            base = b * K
            a0 = nrows[base, pl.ds(0, 16)]
            a1 = nrows[base, pl.ds(16, 16)]
            for k in range(1, K):
                a0 = a0 + nrows[base + k, pl.ds(0, 16)]
                a1 = a1 + nrows[base + k, pl.ds(16, 16)]
            ng_v[pl.ds(b * 16, 16)] = c0 * a0 + c1 * a1
            return 0

        lax.fori_loop(0, C, bbody, 0)
        out_base = (wid * NB + cbase) * 16
        pltpu.sync_copy(uo_v, uo_h.at[pl.ds(out_base, C * 16)])
        pltpu.sync_copy(ng_v, ng_h.at[pl.ds(out_base, C * 16)])
        return 0

    lax.fori_loop(0, NCHUNK, chunk_body, 0)


@jax.jit
def _sc_partials(cidx, oidx, nidx, emb_c_lin, emb_o_lin):
    mesh = plsc.VectorSubcoreMesh(core_axis_name="c", subcore_axis_name="s")
    f32 = jnp.float32
    i32 = jnp.int32
    return pl.kernel(
        _sc_body,
        out_type=(
            jax.ShapeDtypeStruct((B * 16,), f32),
            jax.ShapeDtypeStruct((B * 16,), f32),
        ),
        mesh=mesh,
        compiler_params=pltpu.CompilerParams(use_tc_tiling_on_sc=False),
        scratch_types=[
            pltpu.VMEM((NB,), i32),
            pltpu.VMEM((NB,), i32),
            pltpu.VMEM((NB * K,), i32),
            pltpu.VMEM((C, RW), f32),
            pltpu.VMEM((C, RW), f32),
            pltpu.VMEM((K * C, RW), f32),
            pltpu.VMEM((C * 16,), f32),
            pltpu.VMEM((C * 16,), f32),
            pltpu.SemaphoreType.DMA,
        ],
    )(emb_c_lin, emb_o_lin, cidx, oidx, nidx)


def _tc_body(uo_ref, ng_ref, out_ref):
    uo = uo_ref[...]
    ng = ng_ref[...]
    gi = lax.broadcasted_iota(jnp.int32, (128, 8), 0) // 16
    gj = lax.broadcasted_iota(jnp.int32, (128, 8), 1)
    g = (gi == gj).astype(jnp.float32)
    dn = (((1,), (0,)), ((), ()))
    uos = lax.dot_general(uo, g, dn, preferred_element_type=jnp.float32)
    ngs = lax.dot_general(ng, g, dn, preferred_element_type=jnp.float32)

    def logsig(t):
        return jnp.minimum(t, 0.0) - jnp.log1p(jnp.exp(-jnp.abs(t)))

    loss = logsig(uos) + logsig(-ngs)
    out_ref[0, 0] = -jnp.sum(loss) / jnp.float32(B)


@jax.jit
def _tc_loss(uo2d, ng2d):
    return pl.pallas_call(
        _tc_body,
        out_shape=jax.ShapeDtypeStruct((1, 1), jnp.float32),
        out_specs=pl.BlockSpec(memory_space=pltpu.SMEM),
    )(uo2d, ng2d)


def kernel(center, outside, negative, emb_center, emb_outside):
    cidx = center.reshape(B)
    oidx = outside.reshape(B)
    nidx = negative.reshape(B * K)
    lin_c, lin_o = _sc_relayout(
        emb_center.T, emb_outside.T,
        emb_center[NFULL * 128:].reshape(VREM * D),
        emb_outside[NFULL * 128:].reshape(VREM * D))
    uo, ng = _sc_partials(cidx, oidx, nidx,
                          lin_c.reshape(VPAD, RW), lin_o.reshape(VPAD, RW))
    out = _tc_loss(uo.reshape(B * 16 // 128, 128), ng.reshape(B * 16 // 128, 128))
    return out[0, 0]


# speedup vs baseline: 1.0682x; 1.0033x over previous
import jax
import jax.numpy as jnp
from jax import lax
from jax.experimental import pallas as pl
from jax.experimental.pallas import tpu as pltpu
from jax.experimental.pallas import tpu_sc as plsc

V = 1000000
D = 32
B = 16384
K = 20

NC = 2
NS = 16
NW = NC * NS

NFULL = V // 128
VREM = V - NFULL * 128
VPAD = (NFULL + 1) * 128
RW = 48
LINW = VPAD * RW
GSZ = 512
GN = NFULL * 128 // GSZ
GRP_PER_W = GN // NW + 1

NB = B // NW
C = 64
NCHUNK = NB // C


def _rel_body(tc_h, to_h, tailc_h, tailo_h, linc_h, lino_h,
              stga, stgb, outa, outb, semra, semrb, semw0, semw1):
    i32 = jnp.int32
    wid = lax.axis_index("s") * NC + lax.axis_index("c")
    iota_rw = lax.broadcasted_iota(i32, (16,), 0) * RW
    iota32 = lax.broadcasted_iota(i32, (16,), 0) * 32

    def transpose(stg, ob):
        @plsc.parallel_loop(0, GSZ // 16, unroll=2)
        def gbody(g):
            base = iota_rw + g * (16 * RW)
            for d in range(D):
                vec = stg[d, pl.ds(g * 16, 16)]
                plsc.store_scatter(ob, [base + d], vec)

    def do_table(t_h, tail_h, lin_h):
        def fire_reads(grp, stg, semr):
            for q in range(4):
                pltpu.async_copy(
                    t_h.at[pl.ds(q * 8, 8), pl.ds(grp * GSZ, GSZ)],
                    stg.at[pl.ds(q * 8, 8), :], semr)

        def wait_reads(stg, semr):
            for q in range(4):
                pltpu.make_async_copy(
                    t_h.at[pl.ds(0, 8), pl.ds(0, GSZ)],
                    stg.at[pl.ds(q * 8, 8), :], semr).wait()

        def fire_write(grp, ob, semw):
            pltpu.async_copy(ob, lin_h.at[pl.ds(grp * GSZ * RW, GSZ * RW)], semw)

        def drain_write(ob, semw):
            pltpu.make_async_copy(lin_h.at[pl.ds(0, GSZ * RW)], ob, semw).wait()

        def step(i, grp, stg, semr, ob, semw, pf_grp, pf_stg, pf_semr):
            @pl.when(pf_grp < GN)
            def _():
                fire_reads(pf_grp, pf_stg, pf_semr)

            @pl.when(grp < GN)
            def _():
                wait_reads(stg, semr)

                @pl.when(i > 0)
                def _():
                    drain_write(ob, semw)

                transpose(stg, ob)
                fire_write(grp, ob, semw)

        fire_reads(wid, stga, semra)

        def pair(i, _):
            ga = wid + 64 * i
            gb = ga + 32
            step(i, ga, stga, semra, outa, semw0, gb, stgb, semrb)
            step(i, gb, stgb, semrb, outb, semw1, ga + 64, stga, semra)
            return 0

        lax.fori_loop(0, (GRP_PER_W + 1) // 2, pair, 0)
        drain_write(outa, semw0)
        drain_write(outb, semw1)

        @pl.when(wid == 0)
        def _():
            pltpu.sync_copy(tail_h, outb.at[pl.ds(0, VREM * D)])
            for g in range(VREM // 16):
                for d in range(D):
                    vec = plsc.load_gather(outb, [iota32 + (g * 512 + d)])
                    plsc.store_scatter(outa, [iota_rw + (g * 16 * RW + d)], vec)
            pltpu.sync_copy(outa.at[pl.ds(0, VREM * RW)],
                            lin_h.at[pl.ds(NFULL * 128 * RW, VREM * RW)])

    do_table(tc_h, tailc_h, linc_h)
    do_table(to_h, tailo_h, lino_h)


@jax.jit
def _sc_relayout(tc, to, tailc, tailo):
    mesh = plsc.VectorSubcoreMesh(core_axis_name="c", subcore_axis_name="s")
    f32 = jnp.float32
    return pl.kernel(
        _rel_body,
        out_type=(
            jax.ShapeDtypeStruct((LINW,), f32),
            jax.ShapeDtypeStruct((LINW,), f32),
        ),
        mesh=mesh,
        compiler_params=pltpu.CompilerParams(use_tc_tiling_on_sc=True,
                                             needs_layout_passes=False),
        scratch_types=[
            pltpu.VMEM((D, GSZ), f32),
            pltpu.VMEM((D, GSZ), f32),
            pltpu.VMEM((GSZ * RW,), f32),
            pltpu.VMEM((GSZ * RW,), f32),
            pltpu.SemaphoreType.DMA,
            pltpu.SemaphoreType.DMA,
            pltpu.SemaphoreType.DMA,
            pltpu.SemaphoreType.DMA,
        ],
    )(tc, to, tailc, tailo)


def _sc_body(emb_c, emb_o, cidx_h, oidx_h, nidx_h, uo_h, ng_h,
             cidx_v, oidx_v, nidx_v, crows, orows, nrows, uo_v, ng_v, sem):
    wid = lax.axis_index("s") * NC + lax.axis_index("c")
    pltpu.sync_copy(cidx_h.at[pl.ds(wid * NB, NB)], cidx_v)
    pltpu.sync_copy(oidx_h.at[pl.ds(wid * NB, NB)], oidx_v)
    pltpu.sync_copy(nidx_h.at[pl.ds(wid * NB * K, NB * K)], nidx_v)

    def chunk_body(ch, _):
        cbase = ch * C
        cps = [
            pltpu.async_copy(emb_c.at[cidx_v.at[pl.ds(cbase, C)]], crows, sem),
            pltpu.async_copy(emb_o.at[oidx_v.at[pl.ds(cbase, C)]], orows, sem),
        ]
        for j in range(K):
            cps.append(pltpu.async_copy(
                emb_o.at[nidx_v.at[pl.ds(cbase * K + j * C, C)]],
                nrows.at[pl.ds(j * C, C)], sem))
        for cp in cps:
            cp.wait()

        def bbody(b, _):
            c0 = crows[b, pl.ds(0, 16)]
            c1 = crows[b, pl.ds(16, 16)]
            o0 = orows[b, pl.ds(0, 16)]
            o1 = orows[b, pl.ds(16, 16)]
            uo_v[pl.ds(b * 16, 16)] = c0 * o0 + c1 * o1
            base = b * K
            a0 = nrows[base, pl.ds(0, 16)]
            a1 = nrows[base, pl.ds(16, 16)]
            for k in range(1, K):
                a0 = a0 + nrows[base + k, pl.ds(0, 16)]
                a1 = a1 + nrows[base + k, pl.ds(16, 16)]
            ng_v[pl.ds(b * 16, 16)] = c0 * a0 + c1 * a1
            return 0

        lax.fori_loop(0, C, bbody, 0)
        out_base = (wid * NB + cbase) * 16
        pltpu.sync_copy(uo_v, uo_h.at[pl.ds(out_base, C * 16)])
        pltpu.sync_copy(ng_v, ng_h.at[pl.ds(out_base, C * 16)])
        return 0

    lax.fori_loop(0, NCHUNK, chunk_body, 0)


@jax.jit
def _sc_partials(cidx, oidx, nidx, emb_c_lin, emb_o_lin):
    mesh = plsc.VectorSubcoreMesh(core_axis_name="c", subcore_axis_name="s")
    f32 = jnp.float32
    i32 = jnp.int32
    return pl.kernel(
        _sc_body,
        out_type=(
            jax.ShapeDtypeStruct((B * 16,), f32),
            jax.ShapeDtypeStruct((B * 16,), f32),
        ),
        mesh=mesh,
        compiler_params=pltpu.CompilerParams(use_tc_tiling_on_sc=False),
        scratch_types=[
            pltpu.VMEM((NB,), i32),
            pltpu.VMEM((NB,), i32),
            pltpu.VMEM((NB * K,), i32),
            pltpu.VMEM((C, RW), f32),
            pltpu.VMEM((C, RW), f32),
            pltpu.VMEM((K * C, RW), f32),
            pltpu.VMEM((C * 16,), f32),
            pltpu.VMEM((C * 16,), f32),
            pltpu.SemaphoreType.DMA,
        ],
    )(emb_c_lin, emb_o_lin, cidx, oidx, nidx)


def _tc_body(uo_ref, ng_ref, out_ref):
    uo = uo_ref[...]
    ng = ng_ref[...]
    gi = lax.broadcasted_iota(jnp.int32, (128, 8), 0) // 16
    gj = lax.broadcasted_iota(jnp.int32, (128, 8), 1)
    g = (gi == gj).astype(jnp.float32)
    dn = (((1,), (0,)), ((), ()))
    uos = lax.dot_general(uo, g, dn, preferred_element_type=jnp.float32)
    ngs = lax.dot_general(ng, g, dn, preferred_element_type=jnp.float32)

    def logsig(t):
        return jnp.minimum(t, 0.0) - jnp.log1p(jnp.exp(-jnp.abs(t)))

    loss = logsig(uos) + logsig(-ngs)
    out_ref[0, 0] = -jnp.sum(loss) / jnp.float32(B)


@jax.jit
def _tc_loss(uo2d, ng2d):
    return pl.pallas_call(
        _tc_body,
        out_shape=jax.ShapeDtypeStruct((1, 1), jnp.float32),
        out_specs=pl.BlockSpec(memory_space=pltpu.SMEM),
    )(uo2d, ng2d)


def kernel(center, outside, negative, emb_center, emb_outside):
    cidx = center.reshape(B)
    oidx = outside.reshape(B)
    nidx = negative.reshape(B * K)
    lin_c, lin_o = _sc_relayout(
        emb_center.T, emb_outside.T,
        emb_center[NFULL * 128:].reshape(VREM * D),
        emb_outside[NFULL * 128:].reshape(VREM * D))
    uo, ng = _sc_partials(cidx, oidx, nidx,
                          lin_c.reshape(VPAD, RW), lin_o.reshape(VPAD, RW))
    out = _tc_loss(uo.reshape(B * 16 // 128, 128), ng.reshape(B * 16 // 128, 128))
    return out[0, 0]
